# Initial kernel scaffold; baseline (speedup 1.0000x reference)
import jax
import jax.numpy as jnp
from jax.experimental import pallas as pl


def kernel(text, table, W1, b1, W2, b2):
    raise NotImplementedError("write your pallas kernel here")


# speedup vs baseline: 28.5651x; 28.5651x over previous
import functools

import jax
import jax.numpy as jnp
from jax import lax
from jax.experimental import pallas as pl
from jax.experimental.pallas import tpu as pltpu
from jax.experimental.pallas import tpu_sc as plsc

_EMBED = 32
_SEQ = 200
_BATCH = 4096

_NC = 2
_NS = 16
_NW = _NC * _NS
_NTOK = _BATCH * _SEQ
_PW = _NTOK // _NW
_GI = 128
_NG = _PW // _GI
_KG = 10
_CW = _KG * _GI
_NO = _NG // _KG

@functools.cache
def _make_sc_gather():
    mesh = plsc.VectorSubcoreMesh(core_axis_name="c", subcore_axis_name="s",
                                  num_cores=_NC, num_subcores=_NS)
    return pl.kernel(
        _sc_gather_body,
        out_type=jax.ShapeDtypeStruct((_NTOK, _EMBED), jnp.float32),
        mesh=mesh,
        scratch_types=[
            pltpu.VMEM((_NG, _GI), jnp.int32),
            pltpu.VMEM((2, _CW, _EMBED), jnp.float32),
            pltpu.SemaphoreType.DMA,
            pltpu.SemaphoreType.DMA,
        ],
        compiler_params=pltpu.CompilerParams(use_tc_tiling_on_sc=False),
    )


def _sc_gather_body(idx_hbm, table_hbm, out_hbm, idx_v, rows_v, gsem, wsem):
    wid = lax.axis_index("s") * _NC + lax.axis_index("c")
    row0 = wid * _PW
    pltpu.sync_copy(idx_hbm.at[wid], idx_v)

    def chunk(jj, buf):
        @pl.when(jj >= 2)
        def _():
            pltpu.make_async_copy(
                rows_v.at[buf], out_hbm.at[pl.ds(row0, _CW)], wsem).wait()

        copies = []
        for g in range(_KG):
            copies.append(pltpu.async_copy(
                table_hbm.at[idx_v.at[jj * _KG + g]],
                rows_v.at[buf, pl.ds(g * _GI, _GI), :],
                gsem))
        for c in copies:
            c.wait()
        pltpu.async_copy(
            rows_v.at[buf], out_hbm.at[pl.ds(row0 + jj * _CW, _CW)], wsem)

    def body(it, carry):
        chunk(it * 2, 0)
        chunk(it * 2 + 1, 1)
        return carry

    lax.fori_loop(0, _NO // 2, body, 0)
    pltpu.make_async_copy(rows_v.at[0], out_hbm.at[pl.ds(row0, _CW)], wsem).wait()
    pltpu.make_async_copy(rows_v.at[1], out_hbm.at[pl.ds(row0, _CW)], wsem).wait()


_BM = 256


def _mlp_body(x_ref, w1_ref, b1_ref, w2_ref, b2_ref, o_ref):
    h = jnp.dot(x_ref[...], w1_ref[...], preferred_element_type=jnp.float32)
    h = jnp.maximum(h + b1_ref[...], 0.0)
    o = jnp.dot(h, w2_ref[...], preferred_element_type=jnp.float32) + b2_ref[...]
    o_ref[...] = 1.0 / (1.0 + jnp.exp(-o))


def _tc_mlp(flat, W1, b1, W2, b2):
    k = _SEQ * _EMBED
    return pl.pallas_call(
        _mlp_body,
        grid=(_BATCH // _BM,),
        in_specs=[
            pl.BlockSpec((_BM, k), lambda i: (i, 0)),
            pl.BlockSpec((k, 32), lambda i: (0, 0)),
            pl.BlockSpec((1, 32), lambda i: (0, 0)),
            pl.BlockSpec((32, 1), lambda i: (0, 0)),
            pl.BlockSpec((1, 1), lambda i: (0, 0)),
        ],
        out_specs=pl.BlockSpec((_BM, 1), lambda i: (i, 0)),
        out_shape=jax.ShapeDtypeStruct((_BATCH, 1), jnp.float32),
    )(flat, W1, b1.reshape(1, 32), W2, b2.reshape(1, 1))


def kernel(text, table, W1, b1, W2, b2):
    idx3 = text.astype(jnp.int32).reshape(_NW, _NG, _GI)
    emb = _make_sc_gather()(idx3, table)
    flat = emb.reshape(_BATCH, _SEQ * _EMBED)
    return _tc_mlp(flat, W1, b1, W2, b2)
